# baseline (speedup 1.0000x reference)
import functools

import jax
import jax.numpy as jnp
from jax import lax
from jax.experimental import pallas as pl
from jax.experimental.pallas import tpu as pltpu
from jax.experimental.pallas import tpu_sc as plsc

TILE = 16
N_TILES_X = 64
N_TILES_Y = 64
N_TILES = N_TILES_X * N_TILES_Y
N_CH = 6
HIST = N_CH * N_TILES

NC, NS, L = 2, 16, 16
NW = NC * NS

N_POINTS = 1_000_000
CHUNK_P = 8_000
N_CHUNKS = N_POINTS // CHUNK_P
CHUNK_V = CHUNK_P // L
UNROLL = 10

TPW = N_TILES // NW


@functools.cache
def _build():
    mesh = plsc.VectorSubcoreMesh(
        core_axis_name="c", subcore_axis_name="s", num_cores=NC, num_subcores=NS
    )

    @functools.partial(
        pl.kernel,
        out_type=jax.ShapeDtypeStruct((NW, N_CH, N_TILES), jnp.float32),
        mesh=mesh,
        scratch_types=(
            [pltpu.VMEM((CHUNK_P,), jnp.int32)]
            + [pltpu.VMEM((CHUNK_P,), jnp.float32)] * 5
            + [pltpu.VMEM((CHUNK_P,), jnp.int32)]
            + [pltpu.VMEM((CHUNK_P,), jnp.float32)] * 5
            + [
                pltpu.VMEM((HIST,), jnp.float32),
                pltpu.SemaphoreType.DMA,
                pltpu.SemaphoreType.DMA,
            ]
        ),
        compiler_params=pltpu.CompilerParams(needs_layout_passes=False),
    )
    def _accumulate(t_hbm, r_hbm, g_hbm, b_hbm, a_hbm, d_hbm, part_hbm,
                    t0, r0, g0, b0, a0, d0,
                    t1, r1, g1, b1, a1, d1,
                    hist, sem0, sem1):
        wid = lax.axis_index("s") * NC + lax.axis_index("c")
        hbms = (t_hbm, r_hbm, g_hbm, b_hbm, a_hbm, d_hbm)
        bufs = ((t0, r0, g0, b0, a0, d0), (t1, r1, g1, b1, a1, d1))

        zeros_f = jnp.zeros((L,), jnp.float32)
        ones_f = jnp.ones((L,), jnp.float32)

        def zero_body(j, carry):
            for jj in range(8):
                hist[pl.ds((j * 8 + jj) * L, L)] = zeros_f
            return carry

        lax.fori_loop(0, HIST // (L * 8), zero_body, 0)

        n_k = (N_CHUNKS - wid + NW - 1) // NW

        def start(k, slot, sem):
            base = (wid + k * NW) * CHUNK_P
            for hbm, buf in zip(hbms, bufs[slot]):
                pltpu.async_copy(hbm.at[pl.ds(base, CHUNK_P)], buf, sem)

        def drain(slot, sem):
            for hbm, buf in zip(hbms, bufs[slot]):
                pltpu.make_async_copy(hbm.at[pl.ds(0, CHUNK_P)], buf, sem).wait()

        def process(slot):
            t_v, r_v, g_v, b_v, a_v, d_v = bufs[slot]

            def vec_body(i, icarry):
                for ii in range(UNROLL):
                    sl = pl.ds((i * UNROLL + ii) * L, L)
                    tid = t_v[sl]
                    r = r_v[sl]
                    g = g_v[sl]
                    b = b_v[sl]
                    a = a_v[sl]
                    d = d_v[sl]
                    plsc.addupdate_scatter(hist, [tid], ones_f)
                    plsc.addupdate_scatter(hist, [tid + N_TILES], a)
                    plsc.addupdate_scatter(hist, [tid + 2 * N_TILES], a * r)
                    plsc.addupdate_scatter(hist, [tid + 3 * N_TILES], a * g)
                    plsc.addupdate_scatter(hist, [tid + 4 * N_TILES], a * b)
                    plsc.addupdate_scatter(hist, [tid + 5 * N_TILES], a * d)
                return icarry

            lax.fori_loop(0, CHUNK_V // UNROLL, vec_body, 0)

        @pl.when(n_k > 0)
        def _():
            start(0, 0, sem0)

        def chunk_body(k, carry):
            is_even = lax.rem(k, 2) == 0
            nxt = k + 1

            @pl.when(jnp.logical_and(nxt < n_k, is_even))
            def _():
                start(nxt, 1, sem1)

            @pl.when(jnp.logical_and(nxt < n_k, jnp.logical_not(is_even)))
            def _():
                start(nxt, 0, sem0)

            @pl.when(is_even)
            def _():
                drain(0, sem0)
                process(0)

            @pl.when(jnp.logical_not(is_even))
            def _():
                drain(1, sem1)
                process(1)

            return carry

        lax.fori_loop(0, n_k, chunk_body, 0)

        for c in range(N_CH):
            pltpu.sync_copy(hist.at[pl.ds(c * N_TILES, N_TILES)], part_hbm.at[wid, c])

    @functools.partial(
        pl.kernel,
        out_type=jax.ShapeDtypeStruct((N_TILES * 5,), jnp.float32),
        mesh=mesh,
        scratch_types=[
            pltpu.VMEM((N_CH, NW, TPW), jnp.float32),
            pltpu.VMEM((TPW * 5,), jnp.float32),
            pltpu.SemaphoreType.DMA,
        ],
        compiler_params=pltpu.CompilerParams(needs_layout_passes=False),
    )
    def _finalize(part_hbm, out_hbm, buf, outbuf, sem):
        wid = lax.axis_index("s") * NC + lax.axis_index("c")
        base = wid * TPW
        for c in range(N_CH):
            pltpu.async_copy(part_hbm.at[:, c, pl.ds(base, TPW)], buf.at[c], sem)
        for c in range(N_CH):
            pltpu.make_async_copy(
                part_hbm.at[:, c, pl.ds(base, TPW)], buf.at[c], sem
            ).wait()

        iota = lax.iota(jnp.int32, L)

        for j in range(TPW // L):
            ds_j = pl.ds(j * L, L)

            def red(c):
                def body(p, s):
                    return s + buf[c, p, ds_j]
                return lax.fori_loop(1, NW, body, buf[c, 0, ds_j])

            cnt = red(0)
            asum = red(1)
            rsum = red(2)
            gsum = red(3)
            bsum = red(4)
            dsum = red(5)
            recip = 1.0 / (asum + 1e-6)
            rows5 = (iota + j * L) * 5
            plsc.store_scatter(outbuf, [rows5], rsum * recip)
            plsc.store_scatter(outbuf, [rows5 + 1], gsum * recip)
            plsc.store_scatter(outbuf, [rows5 + 2], bsum * recip)
            plsc.store_scatter(outbuf, [rows5 + 3], dsum * recip)
            plsc.store_scatter(outbuf, [rows5 + 4], cnt)

        pltpu.sync_copy(outbuf, out_hbm.at[pl.ds(base * 5, TPW * 5)])

    def _prep_body(uvt_ref, colt_ref, tid_ref, r_ref, g_ref, b_ref):
        u = uvt_ref[0, :]
        v = uvt_ref[1, :]
        tu = (u * (1.0 / TILE)).astype(jnp.int32)
        tv = (v * (1.0 / TILE)).astype(jnp.int32)
        tid_ref[:] = ((tv << 6) | tu) & (N_TILES - 1)
        r_ref[:] = colt_ref[0, :]
        g_ref[:] = colt_ref[1, :]
        b_ref[:] = colt_ref[2, :]

    _prep = pl.pallas_call(
        _prep_body,
        out_shape=[
            jax.ShapeDtypeStruct((N_POINTS,), jnp.int32),
            jax.ShapeDtypeStruct((N_POINTS,), jnp.float32),
            jax.ShapeDtypeStruct((N_POINTS,), jnp.float32),
            jax.ShapeDtypeStruct((N_POINTS,), jnp.float32),
        ],
    )

    return _prep, _accumulate, _finalize


def kernel(point_uv, point_alpha, point_color, point_depth):
    prep, accumulate, finalize = _build()
    tid, r, g, b = prep(point_uv.T, point_color.T)
    part = accumulate(tid, r, g, b, point_alpha, point_depth)
    out_flat = finalize(part)
    return out_flat.reshape(N_TILES, 5)


# speedup vs baseline: 1.0372x; 1.0372x over previous
import functools

import jax
import jax.numpy as jnp
from jax import lax
from jax.experimental import pallas as pl
from jax.experimental.pallas import tpu as pltpu
from jax.experimental.pallas import tpu_sc as plsc

TILE = 16
N_TILES_X = 64
N_TILES_Y = 64
N_TILES = N_TILES_X * N_TILES_Y
N_CH = 6
HIST = N_CH * N_TILES

NC, NS, L = 2, 16, 16
NW = NC * NS

N_POINTS = 1_000_000
CHUNK_P = 4_000
N_CHUNKS = N_POINTS // CHUNK_P
CHUNK_V = CHUNK_P // L
UNROLL = 5

TPW = N_TILES // NW


@functools.cache
def _build():
    mesh = plsc.VectorSubcoreMesh(
        core_axis_name="c", subcore_axis_name="s", num_cores=NC, num_subcores=NS
    )

    @functools.partial(
        pl.kernel,
        out_type=jax.ShapeDtypeStruct((NW, N_CH, N_TILES), jnp.float32),
        mesh=mesh,
        scratch_types=(
            [pltpu.VMEM((CHUNK_P,), jnp.int32)]
            + [pltpu.VMEM((CHUNK_P,), jnp.float32)] * 5
            + [pltpu.VMEM((CHUNK_P,), jnp.int32)]
            + [pltpu.VMEM((CHUNK_P,), jnp.float32)] * 5
            + [
                pltpu.VMEM((HIST,), jnp.float32),
                pltpu.SemaphoreType.DMA,
                pltpu.SemaphoreType.DMA,
            ]
        ),
        compiler_params=pltpu.CompilerParams(needs_layout_passes=False),
    )
    def _accumulate(t_hbm, r_hbm, g_hbm, b_hbm, a_hbm, d_hbm, part_hbm,
                    t0, r0, g0, b0, a0, d0,
                    t1, r1, g1, b1, a1, d1,
                    hist, sem0, sem1):
        wid = lax.axis_index("s") * NC + lax.axis_index("c")
        hbms = (t_hbm, r_hbm, g_hbm, b_hbm, a_hbm, d_hbm)
        bufs = ((t0, r0, g0, b0, a0, d0), (t1, r1, g1, b1, a1, d1))

        zeros_f = jnp.zeros((L,), jnp.float32)
        ones_f = jnp.ones((L,), jnp.float32)

        def zero_body(j, carry):
            for jj in range(8):
                hist[pl.ds((j * 8 + jj) * L, L)] = zeros_f
            return carry

        lax.fori_loop(0, HIST // (L * 8), zero_body, 0)

        n_k = (N_CHUNKS - wid + NW - 1) // NW

        def start(k, slot, sem):
            base = (wid + k * NW) * CHUNK_P
            for hbm, buf in zip(hbms, bufs[slot]):
                pltpu.async_copy(hbm.at[pl.ds(base, CHUNK_P)], buf, sem)

        def drain(slot, sem):
            for hbm, buf in zip(hbms, bufs[slot]):
                pltpu.make_async_copy(hbm.at[pl.ds(0, CHUNK_P)], buf, sem).wait()

        def process(slot):
            t_v, r_v, g_v, b_v, a_v, d_v = bufs[slot]

            def vec_body(i, icarry):
                for ii in range(UNROLL):
                    sl = pl.ds((i * UNROLL + ii) * L, L)
                    tid = t_v[sl]
                    r = r_v[sl]
                    g = g_v[sl]
                    b = b_v[sl]
                    a = a_v[sl]
                    d = d_v[sl]
                    plsc.addupdate_scatter(hist, [tid], ones_f)
                    plsc.addupdate_scatter(hist, [tid + N_TILES], a)
                    plsc.addupdate_scatter(hist, [tid + 2 * N_TILES], a * r)
                    plsc.addupdate_scatter(hist, [tid + 3 * N_TILES], a * g)
                    plsc.addupdate_scatter(hist, [tid + 4 * N_TILES], a * b)
                    plsc.addupdate_scatter(hist, [tid + 5 * N_TILES], a * d)
                return icarry

            lax.fori_loop(0, CHUNK_V // UNROLL, vec_body, 0)

        @pl.when(n_k > 0)
        def _():
            start(0, 0, sem0)

        def chunk_body(k, carry):
            is_even = lax.rem(k, 2) == 0
            nxt = k + 1

            @pl.when(jnp.logical_and(nxt < n_k, is_even))
            def _():
                start(nxt, 1, sem1)

            @pl.when(jnp.logical_and(nxt < n_k, jnp.logical_not(is_even)))
            def _():
                start(nxt, 0, sem0)

            @pl.when(is_even)
            def _():
                drain(0, sem0)
                process(0)

            @pl.when(jnp.logical_not(is_even))
            def _():
                drain(1, sem1)
                process(1)

            return carry

        lax.fori_loop(0, n_k, chunk_body, 0)

        for c in range(N_CH):
            pltpu.sync_copy(hist.at[pl.ds(c * N_TILES, N_TILES)], part_hbm.at[wid, c])

    @functools.partial(
        pl.kernel,
        out_type=jax.ShapeDtypeStruct((N_TILES * 5,), jnp.float32),
        mesh=mesh,
        scratch_types=[
            pltpu.VMEM((N_CH, NW, TPW), jnp.float32),
            pltpu.VMEM((TPW * 5,), jnp.float32),
            pltpu.SemaphoreType.DMA,
        ],
        compiler_params=pltpu.CompilerParams(needs_layout_passes=False),
    )
    def _finalize(part_hbm, out_hbm, buf, outbuf, sem):
        wid = lax.axis_index("s") * NC + lax.axis_index("c")
        base = wid * TPW
        for c in range(N_CH):
            pltpu.async_copy(part_hbm.at[:, c, pl.ds(base, TPW)], buf.at[c], sem)
        for c in range(N_CH):
            pltpu.make_async_copy(
                part_hbm.at[:, c, pl.ds(base, TPW)], buf.at[c], sem
            ).wait()

        iota = lax.iota(jnp.int32, L)

        for j in range(TPW // L):
            ds_j = pl.ds(j * L, L)

            def red(c):
                def body(p, s):
                    return s + buf[c, p, ds_j]
                return lax.fori_loop(1, NW, body, buf[c, 0, ds_j])

            cnt = red(0)
            asum = red(1)
            rsum = red(2)
            gsum = red(3)
            bsum = red(4)
            dsum = red(5)
            recip = 1.0 / (asum + 1e-6)
            rows5 = (iota + j * L) * 5
            plsc.store_scatter(outbuf, [rows5], rsum * recip)
            plsc.store_scatter(outbuf, [rows5 + 1], gsum * recip)
            plsc.store_scatter(outbuf, [rows5 + 2], bsum * recip)
            plsc.store_scatter(outbuf, [rows5 + 3], dsum * recip)
            plsc.store_scatter(outbuf, [rows5 + 4], cnt)

        pltpu.sync_copy(outbuf, out_hbm.at[pl.ds(base * 5, TPW * 5)])

    def _prep_body(uvt_ref, colt_ref, tid_ref, r_ref, g_ref, b_ref):
        u = uvt_ref[0, :]
        v = uvt_ref[1, :]
        tu = (u * (1.0 / TILE)).astype(jnp.int32)
        tv = (v * (1.0 / TILE)).astype(jnp.int32)
        tid_ref[:] = ((tv << 6) | tu) & (N_TILES - 1)
        r_ref[:] = colt_ref[0, :]
        g_ref[:] = colt_ref[1, :]
        b_ref[:] = colt_ref[2, :]

    PBLK = 131_072
    _prep = pl.pallas_call(
        _prep_body,
        grid=((N_POINTS + PBLK - 1) // PBLK,),
        in_specs=[
            pl.BlockSpec((2, PBLK), lambda i: (0, i)),
            pl.BlockSpec((3, PBLK), lambda i: (0, i)),
        ],
        out_specs=[
            pl.BlockSpec((PBLK,), lambda i: (i,)),
            pl.BlockSpec((PBLK,), lambda i: (i,)),
            pl.BlockSpec((PBLK,), lambda i: (i,)),
            pl.BlockSpec((PBLK,), lambda i: (i,)),
        ],
        out_shape=[
            jax.ShapeDtypeStruct((N_POINTS,), jnp.int32),
            jax.ShapeDtypeStruct((N_POINTS,), jnp.float32),
            jax.ShapeDtypeStruct((N_POINTS,), jnp.float32),
            jax.ShapeDtypeStruct((N_POINTS,), jnp.float32),
        ],
    )

    return _prep, _accumulate, _finalize


def kernel(point_uv, point_alpha, point_color, point_depth):
    prep, accumulate, finalize = _build()
    tid, r, g, b = prep(point_uv.T, point_color.T)
    part = accumulate(tid, r, g, b, point_alpha, point_depth)
    out_flat = finalize(part)
    return out_flat.reshape(N_TILES, 5)
